# baseline (speedup 1.0000x reference)
import functools

import jax
import jax.numpy as jnp
from jax import lax
from jax.experimental import pallas as pl
from jax.experimental.pallas import tpu as pltpu
from jax.experimental.pallas import tpu_sc as plsc

N_TOK = 8192
D = 1024
E = 16
F = 256
M = 2 * N_TOK
BM = 256
NB = M // BM + E
CAP = NB * BM

NC = 2
NS = 16
NW = NC * NS
L = 16


_GTB = 512


def _gating_body(x_ref, wg_ref, i1_ref, i2_ref, g1_ref, g2_ref):
    logits = jnp.dot(x_ref[...], wg_ref[...], preferred_element_type=jnp.float32)
    lane = lax.broadcasted_iota(jnp.int32, logits.shape, 1)
    m1 = jnp.max(logits, axis=1, keepdims=True)
    i1 = jnp.min(jnp.where(logits == m1, lane, E), axis=1)
    masked = jnp.where(lane == i1[:, None], -jnp.inf, logits)
    m2 = jnp.max(masked, axis=1, keepdims=True)
    i2 = jnp.min(jnp.where(masked == m2, lane, E), axis=1)
    g1 = 1.0 / (1.0 + jnp.exp(m2[:, 0] - m1[:, 0]))
    i1_ref[0, 0, :] = i1
    i2_ref[0, 0, :] = i2
    g1_ref[0, 0, :] = g1
    g2_ref[0, 0, :] = 1.0 - g1


def _gating(x, w_gate):
    nb = N_TOK // _GTB
    out_sd = jax.ShapeDtypeStruct((nb, 1, _GTB), jnp.int32)
    out_sdf = jax.ShapeDtypeStruct((nb, 1, _GTB), jnp.float32)
    i1, i2, g1, g2 = pl.pallas_call(
        _gating_body,
        grid=(nb,),
        in_specs=[
            pl.BlockSpec((_GTB, D), lambda b: (b, 0)),
            pl.BlockSpec((D, E), lambda b: (0, 0)),
        ],
        out_specs=[pl.BlockSpec((1, 1, _GTB), lambda b: (b, 0, 0))] * 4,
        out_shape=[out_sd, out_sd, out_sdf, out_sdf],
    )(x, w_gate)
    rs = lambda a: a.reshape(N_TOK)
    return rs(i1), rs(i2), rs(g1), rs(g2)


def _route(i1, i2, g1, g2):
    e_flat = jnp.stack([i1, i2], axis=1).reshape(M)
    gates_flat = jnp.stack([g1, g2], axis=1).reshape(M)
    order = jnp.argsort(e_flat, stable=True).astype(jnp.int32)
    es = e_flat[order]
    counts = jnp.zeros((E,), jnp.int32).at[e_flat].add(1)
    base_raw = jnp.cumsum(counts) - counts
    cap_al = ((counts + BM - 1) // BM) * BM
    base_al = (jnp.cumsum(cap_al) - cap_al).astype(jnp.int32)
    rank = jnp.arange(M, dtype=jnp.int32) - base_raw[es]
    pos = base_al[es] + rank
    sorted_tok = jnp.zeros((CAP,), jnp.int32).at[pos].set(
        (order // 2).astype(jnp.int32))
    sorted_gate = jnp.zeros((CAP,), jnp.float32).at[pos].set(gates_flat[order])
    pos_pair = jnp.zeros((M,), jnp.int32).at[order].set(pos)
    p1, p2 = pos_pair[0::2], pos_pair[1::2]
    bs = jnp.arange(NB, dtype=jnp.int32) * BM
    block_expert = (jnp.sum(base_al[None, :] <= bs[:, None], axis=1) - 1
                    ).astype(jnp.int32)
    return sorted_tok, sorted_gate, p1, p2, block_expert


_GRW = CAP // NW
_GCH = 40
_GNC = _GRW // _GCH


def _gather_body(x_hbm, idx_hbm, out_hbm, idx_v, r0, r1, g0, g1, w0, w1):
    wid = lax.axis_index("s") * NC + lax.axis_index("c")
    base = wid * _GRW
    pltpu.sync_copy(idx_hbm.at[pl.ds(base, _GRW)], idx_v)
    rows = (r0, r1)
    gsem = (g0, g1)
    wsem = (w0, w1)

    def fire(c, b):
        return [pltpu.async_copy(
            x_hbm.at[idx_v.at[pl.ds(c * _GCH + j * 8, 8)]],
            rows[b].at[pl.ds(j * 8, 8)], gsem[b])
            for j in range(_GCH // 8)]

    gh = [fire(0, 0), None]
    wh = [None, None]
    for c in range(_GNC):
        b = c & 1
        if c + 1 < _GNC:
            if wh[1 - b] is not None:
                wh[1 - b].wait()
            gh[1 - b] = fire(c + 1, 1 - b)
        for h in gh[b]:
            h.wait()
        wh[b] = pltpu.async_copy(
            rows[b], out_hbm.at[pl.ds(base + c * _GCH, _GCH)], wsem[b])
    wh[(_GNC - 1) & 1].wait()
    if wh[_GNC & 1] is not None:
        wh[_GNC & 1].wait()


def _gather(x3, sorted_tok):
    return pl.kernel(
        _gather_body,
        out_type=jax.ShapeDtypeStruct((CAP, 8, 128), jnp.float32),
        mesh=plsc.VectorSubcoreMesh(core_axis_name="c", subcore_axis_name="s"),
        scratch_types=[
            pltpu.VMEM((_GRW,), jnp.int32),
            pltpu.VMEM((_GCH, 8, 128), jnp.float32),
            pltpu.VMEM((_GCH, 8, 128), jnp.float32),
            pltpu.SemaphoreType.DMA,
            pltpu.SemaphoreType.DMA,
            pltpu.SemaphoreType.DMA,
            pltpu.SemaphoreType.DMA,
        ],
    )(x3, sorted_tok)


def _gmm_body(be_ref, xs_ref, wd_ref, bd_ref, wu_ref, bu_ref, sg_ref, ys_ref):
    xs = xs_ref[...].reshape(BM, D)
    h = jnp.dot(xs, wd_ref[0], preferred_element_type=jnp.float32)
    h = jnp.maximum(h + bd_ref[0], 0.0)
    y = jnp.dot(h, wu_ref[0], preferred_element_type=jnp.float32)
    sg = sg_ref[0, 0, :].reshape(BM, 1)
    ys_ref[...] = (y + bu_ref[0]) * (0.5 * sg)


def _gmm(block_expert, xs, Wd, bd, Wu, bu, sorted_gate):
    grid_spec = pltpu.PrefetchScalarGridSpec(
        num_scalar_prefetch=1,
        grid=(NB,),
        in_specs=[
            pl.BlockSpec((BM, 8, 128), lambda b, be: (b, 0, 0)),
            pl.BlockSpec((1, D, F), lambda b, be: (be[b], 0, 0)),
            pl.BlockSpec((1, 1, F), lambda b, be: (be[b], 0, 0)),
            pl.BlockSpec((1, F, D), lambda b, be: (be[b], 0, 0)),
            pl.BlockSpec((1, 1, D), lambda b, be: (be[b], 0, 0)),
            pl.BlockSpec((1, 1, BM), lambda b, be: (b, 0, 0)),
        ],
        out_specs=pl.BlockSpec((BM, D), lambda b, be: (b, 0)),
    )
    return pl.pallas_call(
        _gmm_body,
        grid_spec=grid_spec,
        out_shape=jax.ShapeDtypeStruct((CAP, D), jnp.float32),
    )(block_expert, xs, Wd, bd.reshape(E, 1, F), Wu, bu.reshape(E, 1, D),
      sorted_gate.reshape(NB, 1, BM))


_CTW = N_TOK // NW
_CCH = 16
_CNC = _CTW // _CCH


def _combine_body(ys_hbm, p1_hbm, p2_hbm, out_hbm,
                  p1_v, p2_v, a0, a1, b0, b1, ga0, ga1, gb0, gb1, w0, w1):
    wid = lax.axis_index("s") * NC + lax.axis_index("c")
    base = wid * _CTW
    pltpu.sync_copy(p1_hbm.at[pl.ds(base, _CTW)], p1_v)
    pltpu.sync_copy(p2_hbm.at[pl.ds(base, _CTW)], p2_v)
    ra = (a0, a1)
    rb = (b0, b1)
    gsa = (ga0, ga1)
    gsb = (gb0, gb1)
    wsem = (w0, w1)

    def fire(c, k):
        sl = pl.ds(c * _CCH, _CCH)
        return (pltpu.async_copy(ys_hbm.at[p1_v.at[sl]], ra[k], gsa[k]),
                pltpu.async_copy(ys_hbm.at[p2_v.at[sl]], rb[k], gsb[k]))

    gh = [fire(0, 0), None]
    wh = [None, None]
    for c in range(_CNC):
        k = c & 1
        if c + 1 < _CNC:
            if wh[1 - k] is not None:
                wh[1 - k].wait()
            gh[1 - k] = fire(c + 1, 1 - k)
        gh[k][0].wait()
        gh[k][1].wait()

        def tok(t, carry, k=k):
            for v in range(D // L):
                sl = pl.ds(v * L, L)
                ra[k][t, sl] = ra[k][t, sl] + rb[k][t, sl]
            return carry

        lax.fori_loop(0, _CCH, tok, 0)
        wh[k] = pltpu.async_copy(
            ra[k], out_hbm.at[pl.ds(base + c * _CCH, _CCH)], wsem[k])
    wh[(_CNC - 1) & 1].wait()
    if wh[_CNC & 1] is not None:
        wh[_CNC & 1].wait()


def _combine(ys, p1, p2):
    return pl.kernel(
        _combine_body,
        out_type=jax.ShapeDtypeStruct((N_TOK, D), jnp.float32),
        mesh=plsc.VectorSubcoreMesh(core_axis_name="c", subcore_axis_name="s"),
        scratch_types=[
            pltpu.VMEM((_CTW,), jnp.int32),
            pltpu.VMEM((_CTW,), jnp.int32),
            pltpu.VMEM((_CCH, D), jnp.float32),
            pltpu.VMEM((_CCH, D), jnp.float32),
            pltpu.VMEM((_CCH, D), jnp.float32),
            pltpu.VMEM((_CCH, D), jnp.float32),
            pltpu.SemaphoreType.DMA,
            pltpu.SemaphoreType.DMA,
            pltpu.SemaphoreType.DMA,
            pltpu.SemaphoreType.DMA,
            pltpu.SemaphoreType.DMA,
            pltpu.SemaphoreType.DMA,
        ],
    )(ys, p1, p2)


def kernel(x, w_gate, Wd, bd, Wu, bu):
    i1, i2, g1, g2 = _gating(x, w_gate)
    sorted_tok, sorted_gate, p1, p2, block_expert = _route(i1, i2, g1, g2)
    xs = _gather(x.reshape(N_TOK, 8, 128), sorted_tok)
    ys = _gmm(block_expert, xs, Wd, bd, Wu, bu, sorted_gate)
    return _combine(ys, p1, p2)


# speedup vs baseline: 2.2557x; 2.2557x over previous
import functools

import jax
import jax.numpy as jnp
from jax import lax
from jax.experimental import pallas as pl
from jax.experimental.pallas import tpu as pltpu
from jax.experimental.pallas import tpu_sc as plsc

N_TOK = 8192
D = 1024
E = 16
F = 256
M = 2 * N_TOK
BM = 256
NB = M // BM + E
CAP = NB * BM

NC = 2
NS = 16
NW = NC * NS
L = 16
CW = M // NW
TW = N_TOK // NW


_GTB = 512


def _gating_body(x_ref, wg_ref, i1_ref, i2_ref, g1_ref, g2_ref, x3_ref):
    logits = jnp.dot(x_ref[...], wg_ref[...], preferred_element_type=jnp.float32)
    lane = lax.broadcasted_iota(jnp.int32, logits.shape, 1)
    m1 = jnp.max(logits, axis=1, keepdims=True)
    i1 = jnp.min(jnp.where(logits == m1, lane, E), axis=1)
    masked = jnp.where(lane == i1[:, None], -jnp.inf, logits)
    m2 = jnp.max(masked, axis=1, keepdims=True)
    i2 = jnp.min(jnp.where(masked == m2, lane, E), axis=1)
    g1 = 1.0 / (1.0 + jnp.exp(m2[:, 0] - m1[:, 0]))
    i1_ref[0, 0, :] = i1
    i2_ref[0, 0, :] = i2
    g1_ref[0, 0, :] = g1
    g2_ref[0, 0, :] = 1.0 - g1
    x3_ref[...] = x_ref[...].reshape(_GTB, 8, 128)


def _gating(x, w_gate):
    nb = N_TOK // _GTB
    out_sd = jax.ShapeDtypeStruct((nb, 1, _GTB), jnp.int32)
    out_sdf = jax.ShapeDtypeStruct((nb, 1, _GTB), jnp.float32)
    i1, i2, g1, g2, x3 = pl.pallas_call(
        _gating_body,
        grid=(nb,),
        in_specs=[
            pl.BlockSpec((_GTB, D), lambda b: (b, 0)),
            pl.BlockSpec((D, E), lambda b: (0, 0)),
        ],
        out_specs=[pl.BlockSpec((1, 1, _GTB), lambda b: (b, 0, 0))] * 4
        + [pl.BlockSpec((_GTB, 8, 128), lambda b: (b, 0, 0))],
        out_shape=[out_sd, out_sd, out_sdf, out_sdf,
                   jax.ShapeDtypeStruct((N_TOK, 8, 128), jnp.float32)],
    )(x, w_gate)
    rs = lambda a: a.reshape(N_TOK)
    return rs(i1), rs(i2), rs(g1), rs(g2), x3


def _vreg_hist(v, iota):
    hist = jnp.zeros((L,), jnp.int32)
    for k in range(L):
        rot = jnp.take(v, (iota + k) & (L - 1))
        hist = hist + jnp.where(rot == iota, 1, 0)
    return hist


def _vreg_rank(v, iota):
    rank = jnp.zeros((L,), jnp.int32)
    for k in range(1, L):
        sh = jnp.take(v, jnp.maximum(iota - k, 0))
        rank = rank + jnp.where((iota >= k) & (sh == v), 1, 0)
    return rank


def _hist_body(e_hbm, hist_hbm, ev, hv, sem_unused):
    wid = lax.axis_index("s") * NC + lax.axis_index("c")
    pltpu.sync_copy(e_hbm.at[pl.ds(wid * CW, CW)], ev)
    iota = lax.iota(jnp.int32, L)

    def step(i, acc):
        v = ev[pl.ds(i * L, L)]
        return acc + _vreg_hist(v, iota)

    hv[...] = lax.fori_loop(0, CW // L, step, jnp.zeros((L,), jnp.int32))
    pltpu.sync_copy(hv, hist_hbm.at[wid])


def _hist(e_all):
    return pl.kernel(
        _hist_body,
        out_type=jax.ShapeDtypeStruct((NW, L), jnp.int32),
        mesh=plsc.VectorSubcoreMesh(core_axis_name="c", subcore_axis_name="s"),
        scratch_types=[
            pltpu.VMEM((CW,), jnp.int32),
            pltpu.VMEM((L,), jnp.int32),
            pltpu.SemaphoreType.DMA,
        ],
    )(e_all)


def _pos_body(e_hbm, cb_hbm, pos_hbm, ev, off_v, pv, sem_unused):
    wid = lax.axis_index("s") * NC + lax.axis_index("c")
    pltpu.sync_copy(e_hbm.at[pl.ds(wid * CW, CW)], ev)
    pltpu.sync_copy(cb_hbm.at[wid], off_v)
    iota = lax.iota(jnp.int32, L)

    def step(i, off):
        v = ev[pl.ds(i * L, L)]
        rank = _vreg_rank(v, iota)
        pv[pl.ds(i * L, L)] = jnp.take(off, v) + rank
        return off + _vreg_hist(v, iota)

    lax.fori_loop(0, CW // L, step, off_v[...])
    pltpu.sync_copy(pv, pos_hbm.at[pl.ds(wid * CW, CW)])


def _pos(e_all, chunk_base):
    return pl.kernel(
        _pos_body,
        out_type=jax.ShapeDtypeStruct((M,), jnp.int32),
        mesh=plsc.VectorSubcoreMesh(core_axis_name="c", subcore_axis_name="s"),
        scratch_types=[
            pltpu.VMEM((CW,), jnp.int32),
            pltpu.VMEM((L,), jnp.int32),
            pltpu.VMEM((CW,), jnp.int32),
            pltpu.SemaphoreType.DMA,
        ],
    )(e_all, chunk_base)


_SCH = 32
_SNC = TW // _SCH


def _scatter_body(x_hbm, pos_hbm, out_hbm, i1_v, i2_v, r0, r1,
                  g0, g1, w0, w1, isem):
    wid = lax.axis_index("s") * NC + lax.axis_index("c")
    base = wid * TW
    ih = []
    for c in range(_SNC):
        ih.append(pltpu.async_copy(
            pos_hbm.at[pl.ds(base + c * _SCH, _SCH)], i1_v.at[c], isem))
        ih.append(pltpu.async_copy(
            pos_hbm.at[pl.ds(N_TOK + base + c * _SCH, _SCH)], i2_v.at[c], isem))
    for h in ih:
        h.wait()
    rows = (r0, r1)
    gsem = (g0, g1)
    wsem = (w0, w1)

    def load(c, b):
        return pltpu.async_copy(
            x_hbm.at[pl.ds(base + c * _SCH, _SCH)], rows[b], gsem[b])

    def store(c, b):
        return (pltpu.async_copy(rows[b], out_hbm.at[i1_v.at[c]], wsem[b]),
                pltpu.async_copy(rows[b], out_hbm.at[i2_v.at[c]], wsem[b]))

    gh = [load(0, 0), None]
    wh = [None, None]
    for c in range(_SNC):
        b = c & 1
        if c + 1 < _SNC:
            if wh[1 - b] is not None:
                wh[1 - b][0].wait()
                wh[1 - b][1].wait()
            gh[1 - b] = load(c + 1, 1 - b)
        gh[b].wait()
        wh[b] = store(c, b)
    wh[(_SNC - 1) & 1][0].wait()
    wh[(_SNC - 1) & 1][1].wait()
    if _SNC > 1:
        wh[_SNC & 1][0].wait()
        wh[_SNC & 1][1].wait()


def _scatter(x3, pos):
    return pl.kernel(
        _scatter_body,
        out_type=jax.ShapeDtypeStruct((CAP, 8, 128), jnp.float32),
        mesh=plsc.VectorSubcoreMesh(core_axis_name="c", subcore_axis_name="s"),
        scratch_types=[
            pltpu.VMEM((_SNC, _SCH), jnp.int32),
            pltpu.VMEM((_SNC, _SCH), jnp.int32),
            pltpu.VMEM((_SCH, 8, 128), jnp.float32),
            pltpu.VMEM((_SCH, 8, 128), jnp.float32),
            pltpu.SemaphoreType.DMA,
            pltpu.SemaphoreType.DMA,
            pltpu.SemaphoreType.DMA,
            pltpu.SemaphoreType.DMA,
            pltpu.SemaphoreType.DMA,
        ],
    )(x3, pos)


def _gmm_body(be_ref, xs_ref, wd_ref, bd_ref, wu_ref, bu_ref, ys_ref):
    xs = xs_ref[...].reshape(BM, D)
    h = jnp.dot(xs, wd_ref[0], preferred_element_type=jnp.float32)
    h = jnp.maximum(h + bd_ref[0], 0.0)
    y = jnp.dot(h, wu_ref[0], preferred_element_type=jnp.float32)
    ys_ref[...] = (y + bu_ref[0]) * 0.5


def _gmm(block_expert, xs, Wd, bd, Wu, bu):
    grid_spec = pltpu.PrefetchScalarGridSpec(
        num_scalar_prefetch=1,
        grid=(NB,),
        in_specs=[
            pl.BlockSpec((BM, 8, 128), lambda b, be: (b, 0, 0)),
            pl.BlockSpec((1, D, F), lambda b, be: (be[b], 0, 0)),
            pl.BlockSpec((1, 1, F), lambda b, be: (be[b], 0, 0)),
            pl.BlockSpec((1, F, D), lambda b, be: (be[b], 0, 0)),
            pl.BlockSpec((1, 1, D), lambda b, be: (be[b], 0, 0)),
        ],
        out_specs=pl.BlockSpec((BM, D), lambda b, be: (b, 0)),
    )
    return pl.pallas_call(
        _gmm_body,
        grid_spec=grid_spec,
        out_shape=jax.ShapeDtypeStruct((CAP, D), jnp.float32),
    )(block_expert, xs, Wd, bd.reshape(E, 1, F), Wu, bu.reshape(E, 1, D))


_CCH = 16
_CNC = TW // _CCH


def _combine_body(ys_hbm, pos_hbm, g1_hbm, g2_hbm, out_hbm,
                  p1_v, p2_v, g1_v, g2_v, a0, a1, b0, b1,
                  ga0, ga1, gb0, gb1, w0, w1):
    wid = lax.axis_index("s") * NC + lax.axis_index("c")
    base = wid * TW
    pltpu.sync_copy(pos_hbm.at[pl.ds(base, TW)], p1_v)
    pltpu.sync_copy(pos_hbm.at[pl.ds(N_TOK + base, TW)], p2_v)
    pltpu.sync_copy(g1_hbm.at[pl.ds(base, TW)], g1_v)
    pltpu.sync_copy(g2_hbm.at[pl.ds(base, TW)], g2_v)
    ra = (a0, a1)
    rb = (b0, b1)
    gsa = (ga0, ga1)
    gsb = (gb0, gb1)
    wsem = (w0, w1)
    iota = lax.iota(jnp.int32, L)

    def fire(c, k):
        sl = pl.ds(c * _CCH, _CCH)
        return (pltpu.async_copy(ys_hbm.at[p1_v.at[sl]], ra[k], gsa[k]),
                pltpu.async_copy(ys_hbm.at[p2_v.at[sl]], rb[k], gsb[k]))

    gh = [fire(0, 0), None]
    wh = [None, None]
    for c in range(_CNC):
        k = c & 1
        if c + 1 < _CNC:
            if wh[1 - k] is not None:
                wh[1 - k].wait()
            gh[1 - k] = fire(c + 1, 1 - k)
        gh[k][0].wait()
        gh[k][1].wait()
        g1c = g1_v[pl.ds(c * _CCH, L)]
        g2c = g2_v[pl.ds(c * _CCH, L)]

        def tok(t, carry, k=k, g1c=g1c, g2c=g2c):
            ga = jnp.take(g1c, jnp.zeros((L,), jnp.int32) + t)
            gb = jnp.take(g2c, jnp.zeros((L,), jnp.int32) + t)
            for v in range(D // L):
                sl = pl.ds(v * L, L)
                ra[k][t, sl] = ga * ra[k][t, sl] + gb * rb[k][t, sl]
            return carry

        lax.fori_loop(0, _CCH, tok, 0)
        wh[k] = pltpu.async_copy(
            ra[k], out_hbm.at[pl.ds(base + c * _CCH, _CCH)], wsem[k])
    wh[(_CNC - 1) & 1].wait()
    if wh[_CNC & 1] is not None:
        wh[_CNC & 1].wait()


def _combine(ys, pos, g1, g2):
    return pl.kernel(
        _combine_body,
        out_type=jax.ShapeDtypeStruct((N_TOK, D), jnp.float32),
        mesh=plsc.VectorSubcoreMesh(core_axis_name="c", subcore_axis_name="s"),
        scratch_types=[
            pltpu.VMEM((TW,), jnp.int32),
            pltpu.VMEM((TW,), jnp.int32),
            pltpu.VMEM((TW,), jnp.float32),
            pltpu.VMEM((TW,), jnp.float32),
            pltpu.VMEM((_CCH, D), jnp.float32),
            pltpu.VMEM((_CCH, D), jnp.float32),
            pltpu.VMEM((_CCH, D), jnp.float32),
            pltpu.VMEM((_CCH, D), jnp.float32),
            pltpu.SemaphoreType.DMA,
            pltpu.SemaphoreType.DMA,
            pltpu.SemaphoreType.DMA,
            pltpu.SemaphoreType.DMA,
            pltpu.SemaphoreType.DMA,
            pltpu.SemaphoreType.DMA,
        ],
    )(ys, pos, g1, g2)


def kernel(x, w_gate, Wd, bd, Wu, bu):
    i1, i2, g1, g2, x3 = _gating(x, w_gate)
    e_all = jnp.concatenate([i1, i2])
    hists = _hist(e_all)
    counts = jnp.sum(hists, axis=0)
    cap_al = ((counts + BM - 1) // BM) * BM
    base_al = (jnp.cumsum(cap_al) - cap_al).astype(jnp.int32)
    chunk_base = (base_al[None, :] + jnp.cumsum(hists, axis=0) - hists
                  ).astype(jnp.int32)
    bs = jnp.arange(NB, dtype=jnp.int32) * BM
    block_expert = (jnp.sum(base_al[None, :] <= bs[:, None], axis=1) - 1
                    ).astype(jnp.int32)
    pos = _pos(e_all, chunk_base)
    xs = _scatter(x3, pos)
    ys = _gmm(block_expert, xs, Wd, bd, Wu, bu)
    return _combine(ys, pos, g1, g2)


# speedup vs baseline: 2.2587x; 1.0013x over previous
import functools

import jax
import jax.numpy as jnp
from jax import lax
from jax.experimental import pallas as pl
from jax.experimental.pallas import tpu as pltpu
from jax.experimental.pallas import tpu_sc as plsc

N_TOK = 8192
D = 1024
E = 16
F = 256
M = 2 * N_TOK
BM = 256
NB = M // BM + E
CAP = NB * BM

NC = 2
NS = 16
NW = NC * NS
L = 16
CW = M // NW
TW = N_TOK // NW


_GTB = 512


def _gating_body(x_ref, wg_ref, i1_ref, i2_ref, g1_ref, g2_ref, x3_ref):
    logits = jnp.dot(x_ref[...], wg_ref[...], preferred_element_type=jnp.float32)
    lane = lax.broadcasted_iota(jnp.int32, logits.shape, 1)
    m1 = jnp.max(logits, axis=1, keepdims=True)
    i1 = jnp.min(jnp.where(logits == m1, lane, E), axis=1)
    masked = jnp.where(lane == i1[:, None], -jnp.inf, logits)
    m2 = jnp.max(masked, axis=1, keepdims=True)
    i2 = jnp.min(jnp.where(masked == m2, lane, E), axis=1)
    g1 = 1.0 / (1.0 + jnp.exp(m2[:, 0] - m1[:, 0]))
    i1_ref[0, 0, :] = i1
    i2_ref[0, 0, :] = i2
    g1_ref[0, 0, :] = g1
    g2_ref[0, 0, :] = 1.0 - g1
    x3_ref[...] = x_ref[...].reshape(_GTB, 8, 128)


def _gating(x, w_gate):
    nb = N_TOK // _GTB
    out_sd = jax.ShapeDtypeStruct((nb, 1, _GTB), jnp.int32)
    out_sdf = jax.ShapeDtypeStruct((nb, 1, _GTB), jnp.float32)
    i1, i2, g1, g2, x3 = pl.pallas_call(
        _gating_body,
        grid=(nb,),
        in_specs=[
            pl.BlockSpec((_GTB, D), lambda b: (b, 0)),
            pl.BlockSpec((D, E), lambda b: (0, 0)),
        ],
        out_specs=[pl.BlockSpec((1, 1, _GTB), lambda b: (b, 0, 0))] * 4
        + [pl.BlockSpec((_GTB, 8, 128), lambda b: (b, 0, 0))],
        out_shape=[out_sd, out_sd, out_sdf, out_sdf,
                   jax.ShapeDtypeStruct((N_TOK, 8, 128), jnp.float32)],
    )(x, w_gate)
    rs = lambda a: a.reshape(N_TOK)
    return rs(i1), rs(i2), rs(g1), rs(g2), x3


def _vreg_hist(v, iota):
    hist = jnp.zeros((L,), jnp.int32)
    for k in range(L):
        rot = jnp.take(v, (iota + k) & (L - 1))
        hist = hist + jnp.where(rot == iota, 1, 0)
    return hist


def _vreg_rank(v, iota):
    rank = jnp.zeros((L,), jnp.int32)
    for k in range(1, L):
        sh = jnp.take(v, jnp.maximum(iota - k, 0))
        rank = rank + jnp.where((iota >= k) & (sh == v), 1, 0)
    return rank


def _load_pairs(i1_hbm, i2_hbm, wid, ev1, ev2):
    m = (wid % NS) * CW
    pltpu.sync_copy(i1_hbm.at[pl.ds(m, CW)], ev1)
    pltpu.sync_copy(i2_hbm.at[pl.ds(m, CW)], ev2)


def _hist_body(i1_hbm, i2_hbm, hist_hbm, ev1, ev2, hv, sem_unused):
    wid = lax.axis_index("s") * NC + lax.axis_index("c")
    _load_pairs(i1_hbm, i2_hbm, wid, ev1, ev2)
    iota = lax.iota(jnp.int32, L)
    sel = wid < NS

    def step(i, acc):
        v = jnp.where(sel, ev1[pl.ds(i * L, L)], ev2[pl.ds(i * L, L)])
        return acc + _vreg_hist(v, iota)

    hv[...] = lax.fori_loop(0, CW // L, step, jnp.zeros((L,), jnp.int32))
    pltpu.sync_copy(hv, hist_hbm.at[wid])


def _hist(i1, i2):
    return pl.kernel(
        _hist_body,
        out_type=jax.ShapeDtypeStruct((NW, L), jnp.int32),
        mesh=plsc.VectorSubcoreMesh(core_axis_name="c", subcore_axis_name="s"),
        scratch_types=[
            pltpu.VMEM((CW,), jnp.int32),
            pltpu.VMEM((CW,), jnp.int32),
            pltpu.VMEM((L,), jnp.int32),
            pltpu.SemaphoreType.DMA,
        ],
    )(i1, i2)


def _pos_body(i1_hbm, i2_hbm, hist_hbm, pos_hbm, ev1, ev2, hv, pv, sem_unused):
    wid = lax.axis_index("s") * NC + lax.axis_index("c")
    _load_pairs(i1_hbm, i2_hbm, wid, ev1, ev2)
    pltpu.sync_copy(hist_hbm, hv)
    iota = lax.iota(jnp.int32, L)
    sel = wid < NS

    total = jnp.zeros((L,), jnp.int32)
    partial = jnp.zeros((L,), jnp.int32)
    for w in range(NW):
        row = hv[w, :]
        total = total + row
        partial = partial + jnp.where(wid > w, row, 0)
    cap = ((total + (BM - 1)) >> 8) << 8
    pref = cap
    for k in (1, 2, 4, 8):
        pref = pref + jnp.where(iota >= k, jnp.take(pref, jnp.maximum(iota - k, 0)), 0)
    off0 = (pref - cap) + partial

    def step(i, off):
        v = jnp.where(sel, ev1[pl.ds(i * L, L)], ev2[pl.ds(i * L, L)])
        rank = _vreg_rank(v, iota)
        pv[pl.ds(i * L, L)] = jnp.take(off, v) + rank
        return off + _vreg_hist(v, iota)

    lax.fori_loop(0, CW // L, step, off0)
    pltpu.sync_copy(pv, pos_hbm.at[pl.ds(wid * CW, CW)])


def _pos(i1, i2, hists):
    return pl.kernel(
        _pos_body,
        out_type=jax.ShapeDtypeStruct((M,), jnp.int32),
        mesh=plsc.VectorSubcoreMesh(core_axis_name="c", subcore_axis_name="s"),
        scratch_types=[
            pltpu.VMEM((CW,), jnp.int32),
            pltpu.VMEM((CW,), jnp.int32),
            pltpu.VMEM((NW, L), jnp.int32),
            pltpu.VMEM((CW,), jnp.int32),
            pltpu.SemaphoreType.DMA,
        ],
    )(i1, i2, hists)


_SCH = 32
_SNC = TW // _SCH


def _scatter_body(x_hbm, pos_hbm, out_hbm, i1_v, i2_v, r0, r1,
                  g0, g1, w0, w1, isem):
    wid = lax.axis_index("s") * NC + lax.axis_index("c")
    base = wid * TW
    ih = []
    for c in range(_SNC):
        ih.append(pltpu.async_copy(
            pos_hbm.at[pl.ds(base + c * _SCH, _SCH)], i1_v.at[c], isem))
        ih.append(pltpu.async_copy(
            pos_hbm.at[pl.ds(N_TOK + base + c * _SCH, _SCH)], i2_v.at[c], isem))
    for h in ih:
        h.wait()
    rows = (r0, r1)
    gsem = (g0, g1)
    wsem = (w0, w1)

    def load(c, b):
        return pltpu.async_copy(
            x_hbm.at[pl.ds(base + c * _SCH, _SCH)], rows[b], gsem[b])

    def store(c, b):
        return (pltpu.async_copy(rows[b], out_hbm.at[i1_v.at[c]], wsem[b]),
                pltpu.async_copy(rows[b], out_hbm.at[i2_v.at[c]], wsem[b]))

    gh = [load(0, 0), None]
    wh = [None, None]
    for c in range(_SNC):
        b = c & 1
        if c + 1 < _SNC:
            if wh[1 - b] is not None:
                wh[1 - b][0].wait()
                wh[1 - b][1].wait()
            gh[1 - b] = load(c + 1, 1 - b)
        gh[b].wait()
        wh[b] = store(c, b)
    wh[(_SNC - 1) & 1][0].wait()
    wh[(_SNC - 1) & 1][1].wait()
    if _SNC > 1:
        wh[_SNC & 1][0].wait()
        wh[_SNC & 1][1].wait()


def _scatter(x3, pos):
    return pl.kernel(
        _scatter_body,
        out_type=jax.ShapeDtypeStruct((CAP, 8, 128), jnp.float32),
        mesh=plsc.VectorSubcoreMesh(core_axis_name="c", subcore_axis_name="s"),
        scratch_types=[
            pltpu.VMEM((_SNC, _SCH), jnp.int32),
            pltpu.VMEM((_SNC, _SCH), jnp.int32),
            pltpu.VMEM((_SCH, 8, 128), jnp.float32),
            pltpu.VMEM((_SCH, 8, 128), jnp.float32),
            pltpu.SemaphoreType.DMA,
            pltpu.SemaphoreType.DMA,
            pltpu.SemaphoreType.DMA,
            pltpu.SemaphoreType.DMA,
            pltpu.SemaphoreType.DMA,
        ],
    )(x3, pos)


def _gmm_body(be_ref, xs_ref, wd_ref, bd_ref, wu_ref, bu_ref, ys_ref):
    xs = xs_ref[...].reshape(BM, D)
    h = jnp.dot(xs, wd_ref[0], preferred_element_type=jnp.float32)
    h = jnp.maximum(h + bd_ref[0], 0.0)
    y = jnp.dot(h, wu_ref[0], preferred_element_type=jnp.float32)
    ys_ref[...] = (y + bu_ref[0]) * 0.5


def _gmm(block_expert, xs, Wd, bd, Wu, bu):
    grid_spec = pltpu.PrefetchScalarGridSpec(
        num_scalar_prefetch=1,
        grid=(NB,),
        in_specs=[
            pl.BlockSpec((BM, 8, 128), lambda b, be: (b, 0, 0)),
            pl.BlockSpec((1, D, F), lambda b, be: (be[b], 0, 0)),
            pl.BlockSpec((1, 1, F), lambda b, be: (be[b], 0, 0)),
            pl.BlockSpec((1, F, D), lambda b, be: (be[b], 0, 0)),
            pl.BlockSpec((1, 1, D), lambda b, be: (be[b], 0, 0)),
        ],
        out_specs=pl.BlockSpec((BM, D), lambda b, be: (b, 0)),
    )
    return pl.pallas_call(
        _gmm_body,
        grid_spec=grid_spec,
        out_shape=jax.ShapeDtypeStruct((CAP, D), jnp.float32),
    )(block_expert, xs, Wd, bd.reshape(E, 1, F), Wu, bu.reshape(E, 1, D))


_CCH = 16
_CNC = TW // _CCH


def _combine_body(ys_hbm, pos_hbm, g1_hbm, g2_hbm, out_hbm,
                  p1_v, p2_v, g1_v, g2_v, a0, a1, b0, b1,
                  ga0, ga1, gb0, gb1, w0, w1):
    wid = lax.axis_index("s") * NC + lax.axis_index("c")
    base = wid * TW
    pltpu.sync_copy(pos_hbm.at[pl.ds(base, TW)], p1_v)
    pltpu.sync_copy(pos_hbm.at[pl.ds(N_TOK + base, TW)], p2_v)
    pltpu.sync_copy(g1_hbm.at[pl.ds(base, TW)], g1_v)
    pltpu.sync_copy(g2_hbm.at[pl.ds(base, TW)], g2_v)
    ra = (a0, a1)
    rb = (b0, b1)
    gsa = (ga0, ga1)
    gsb = (gb0, gb1)
    wsem = (w0, w1)
    iota = lax.iota(jnp.int32, L)

    def fire(c, k):
        sl = pl.ds(c * _CCH, _CCH)
        return (pltpu.async_copy(ys_hbm.at[p1_v.at[sl]], ra[k], gsa[k]),
                pltpu.async_copy(ys_hbm.at[p2_v.at[sl]], rb[k], gsb[k]))

    gh = [fire(0, 0), None]
    wh = [None, None]
    for c in range(_CNC):
        k = c & 1
        if c + 1 < _CNC:
            if wh[1 - k] is not None:
                wh[1 - k].wait()
            gh[1 - k] = fire(c + 1, 1 - k)
        gh[k][0].wait()
        gh[k][1].wait()
        g1c = g1_v[pl.ds(c * _CCH, L)]
        g2c = g2_v[pl.ds(c * _CCH, L)]

        def tok(t, carry, k=k, g1c=g1c, g2c=g2c):
            ga = jnp.take(g1c, jnp.zeros((L,), jnp.int32) + t)
            gb = jnp.take(g2c, jnp.zeros((L,), jnp.int32) + t)
            for v in range(D // L):
                sl = pl.ds(v * L, L)
                ra[k][t, sl] = ga * ra[k][t, sl] + gb * rb[k][t, sl]
            return carry

        lax.fori_loop(0, _CCH, tok, 0)
        wh[k] = pltpu.async_copy(
            ra[k], out_hbm.at[pl.ds(base + c * _CCH, _CCH)], wsem[k])
    wh[(_CNC - 1) & 1].wait()
    if wh[_CNC & 1] is not None:
        wh[_CNC & 1].wait()


def _combine(ys, pos, g1, g2):
    return pl.kernel(
        _combine_body,
        out_type=jax.ShapeDtypeStruct((N_TOK, D), jnp.float32),
        mesh=plsc.VectorSubcoreMesh(core_axis_name="c", subcore_axis_name="s"),
        scratch_types=[
            pltpu.VMEM((TW,), jnp.int32),
            pltpu.VMEM((TW,), jnp.int32),
            pltpu.VMEM((TW,), jnp.float32),
            pltpu.VMEM((TW,), jnp.float32),
            pltpu.VMEM((_CCH, D), jnp.float32),
            pltpu.VMEM((_CCH, D), jnp.float32),
            pltpu.VMEM((_CCH, D), jnp.float32),
            pltpu.VMEM((_CCH, D), jnp.float32),
            pltpu.SemaphoreType.DMA,
            pltpu.SemaphoreType.DMA,
            pltpu.SemaphoreType.DMA,
            pltpu.SemaphoreType.DMA,
            pltpu.SemaphoreType.DMA,
            pltpu.SemaphoreType.DMA,
        ],
    )(ys, pos, g1, g2)


def kernel(x, w_gate, Wd, bd, Wu, bu):
    i1, i2, g1, g2, x3 = _gating(x, w_gate)
    hists = _hist(i1, i2)
    counts = jnp.sum(hists, axis=0)
    cap_al = ((counts + BM - 1) // BM) * BM
    base_al = (jnp.cumsum(cap_al) - cap_al).astype(jnp.int32)
    bs = jnp.arange(NB, dtype=jnp.int32) * BM
    block_expert = (jnp.sum(base_al[None, :] <= bs[:, None], axis=1) - 1
                    ).astype(jnp.int32)
    pos = _pos(i1, i2, hists)
    xs = _scatter(x3, pos)
    ys = _gmm(block_expert, xs, Wd, bd, Wu, bu)
    return _combine(ys, pos, g1, g2)


# speedup vs baseline: 2.2631x; 1.0020x over previous
import jax
import jax.numpy as jnp
from jax import lax
from jax.experimental import pallas as pl
from jax.experimental.pallas import tpu as pltpu
from jax.experimental.pallas import tpu_sc as plsc

N_TOK = 8192
D = 1024
E = 16
F = 256
M = 2 * N_TOK
BM = 256
NB = M // BM + E
CAP = NB * BM

NC = 2
NS = 16
NW = NC * NS
L = 16
CW = M // NW
TW = N_TOK // NW


_GTB = 512


def _gating_body(x_ref, wg_ref, i1_ref, i2_ref, g1_ref, g2_ref, x3_ref):
    logits = jnp.dot(x_ref[...], wg_ref[...], preferred_element_type=jnp.float32)
    lane = lax.broadcasted_iota(jnp.int32, logits.shape, 1)
    m1 = jnp.max(logits, axis=1, keepdims=True)
    i1 = jnp.min(jnp.where(logits == m1, lane, E), axis=1)
    masked = jnp.where(lane == i1[:, None], -jnp.inf, logits)
    m2 = jnp.max(masked, axis=1, keepdims=True)
    i2 = jnp.min(jnp.where(masked == m2, lane, E), axis=1)
    g1 = 1.0 / (1.0 + jnp.exp(m2[:, 0] - m1[:, 0]))
    i1_ref[0, 0, :] = i1
    i2_ref[0, 0, :] = i2
    g1_ref[0, 0, :] = g1
    g2_ref[0, 0, :] = 1.0 - g1
    x3_ref[...] = x_ref[...].reshape(_GTB, 8, 128)


def _gating(x, w_gate):
    nb = N_TOK // _GTB
    out_sd = jax.ShapeDtypeStruct((nb, 1, _GTB), jnp.int32)
    out_sdf = jax.ShapeDtypeStruct((nb, 1, _GTB), jnp.float32)
    i1, i2, g1, g2, x3 = pl.pallas_call(
        _gating_body,
        grid=(nb,),
        in_specs=[
            pl.BlockSpec((_GTB, D), lambda b: (b, 0)),
            pl.BlockSpec((D, E), lambda b: (0, 0)),
        ],
        out_specs=[pl.BlockSpec((1, 1, _GTB), lambda b: (b, 0, 0))] * 4
        + [pl.BlockSpec((_GTB, 8, 128), lambda b: (b, 0, 0))],
        out_shape=[out_sd, out_sd, out_sdf, out_sdf,
                   jax.ShapeDtypeStruct((N_TOK, 8, 128), jnp.float32)],
    )(x, w_gate)
    rs = lambda a: a.reshape(N_TOK)
    return rs(i1), rs(i2), rs(g1), rs(g2), x3


def _vreg_hist(v, iota):
    hist = jnp.zeros((L,), jnp.int32)
    for k in range(L):
        rot = jnp.take(v, (iota + k) & (L - 1))
        hist = hist + jnp.where(rot == iota, 1, 0)
    return hist


def _vreg_rank(v, iota):
    rank = jnp.zeros((L,), jnp.int32)
    for k in range(1, L):
        sh = jnp.take(v, jnp.maximum(iota - k, 0))
        rank = rank + jnp.where((iota >= k) & (sh == v), 1, 0)
    return rank


def _load_pairs(i1_hbm, i2_hbm, wid, ev1, ev2):
    m = (wid % NS) * CW
    pltpu.sync_copy(i1_hbm.at[pl.ds(m, CW)], ev1)
    pltpu.sync_copy(i2_hbm.at[pl.ds(m, CW)], ev2)


def _hist_body(i1_hbm, i2_hbm, hist_hbm, ev1, ev2, hv, sem_unused):
    wid = lax.axis_index("s") * NC + lax.axis_index("c")
    _load_pairs(i1_hbm, i2_hbm, wid, ev1, ev2)
    iota = lax.iota(jnp.int32, L)
    sel = wid < NS

    def step(i, acc):
        v = jnp.where(sel, ev1[pl.ds(i * L, L)], ev2[pl.ds(i * L, L)])
        return acc + _vreg_hist(v, iota)

    hv[...] = lax.fori_loop(0, CW // L, step, jnp.zeros((L,), jnp.int32))
    pltpu.sync_copy(hv, hist_hbm.at[wid])


def _hist(i1, i2):
    return pl.kernel(
        _hist_body,
        out_type=jax.ShapeDtypeStruct((NW, L), jnp.int32),
        mesh=plsc.VectorSubcoreMesh(core_axis_name="c", subcore_axis_name="s"),
        scratch_types=[
            pltpu.VMEM((CW,), jnp.int32),
            pltpu.VMEM((CW,), jnp.int32),
            pltpu.VMEM((L,), jnp.int32),
            pltpu.SemaphoreType.DMA,
        ],
    )(i1, i2)


def _pos_body(i1_hbm, i2_hbm, hist_hbm, pos_hbm, ev1, ev2, hv, pv, sem_unused):
    wid = lax.axis_index("s") * NC + lax.axis_index("c")
    _load_pairs(i1_hbm, i2_hbm, wid, ev1, ev2)
    pltpu.sync_copy(hist_hbm, hv)
    iota = lax.iota(jnp.int32, L)
    sel = wid < NS

    total = jnp.zeros((L,), jnp.int32)
    partial = jnp.zeros((L,), jnp.int32)
    for w in range(NW):
        row = hv[w, :]
        total = total + row
        partial = partial + jnp.where(wid > w, row, 0)
    cap = ((total + (BM - 1)) >> 8) << 8
    pref = cap
    for k in (1, 2, 4, 8):
        pref = pref + jnp.where(iota >= k, jnp.take(pref, jnp.maximum(iota - k, 0)), 0)
    off0 = (pref - cap) + partial

    def step(i, off):
        v = jnp.where(sel, ev1[pl.ds(i * L, L)], ev2[pl.ds(i * L, L)])
        rank = _vreg_rank(v, iota)
        pv[pl.ds(i * L, L)] = jnp.take(off, v) + rank
        return off + _vreg_hist(v, iota)

    lax.fori_loop(0, CW // L, step, off0)
    pltpu.sync_copy(pv, pos_hbm.at[pl.ds(wid * CW, CW)])


def _pos(i1, i2, hists):
    return pl.kernel(
        _pos_body,
        out_type=jax.ShapeDtypeStruct((M,), jnp.int32),
        mesh=plsc.VectorSubcoreMesh(core_axis_name="c", subcore_axis_name="s"),
        scratch_types=[
            pltpu.VMEM((CW,), jnp.int32),
            pltpu.VMEM((CW,), jnp.int32),
            pltpu.VMEM((NW, L), jnp.int32),
            pltpu.VMEM((CW,), jnp.int32),
            pltpu.SemaphoreType.DMA,
        ],
    )(i1, i2, hists)


_SCH = 32
_SNC = TW // _SCH


def _scatter_body(x_hbm, pos_hbm, out_hbm, i1_v, i2_v, r0, r1,
                  g0, g1, w0, w1, isem):
    wid = lax.axis_index("s") * NC + lax.axis_index("c")
    base = wid * TW
    ih = []
    for c in range(_SNC):
        ih.append(pltpu.async_copy(
            pos_hbm.at[pl.ds(base + c * _SCH, _SCH)], i1_v.at[c], isem))
        ih.append(pltpu.async_copy(
            pos_hbm.at[pl.ds(N_TOK + base + c * _SCH, _SCH)], i2_v.at[c], isem))
    for h in ih:
        h.wait()
    rows = (r0, r1)
    gsem = (g0, g1)
    wsem = (w0, w1)

    def load(c, b):
        return pltpu.async_copy(
            x_hbm.at[pl.ds(base + c * _SCH, _SCH)], rows[b], gsem[b])

    def store(c, b):
        return (pltpu.async_copy(rows[b], out_hbm.at[i1_v.at[c]], wsem[b]),
                pltpu.async_copy(rows[b], out_hbm.at[i2_v.at[c]], wsem[b]))

    gh = [load(0, 0), None]
    wh = [None, None]
    for c in range(_SNC):
        b = c & 1
        if c + 1 < _SNC:
            if wh[1 - b] is not None:
                wh[1 - b][0].wait()
                wh[1 - b][1].wait()
            gh[1 - b] = load(c + 1, 1 - b)
        gh[b].wait()
        wh[b] = store(c, b)
    wh[(_SNC - 1) & 1][0].wait()
    wh[(_SNC - 1) & 1][1].wait()
    if _SNC > 1:
        wh[_SNC & 1][0].wait()
        wh[_SNC & 1][1].wait()


def _scatter(x3, pos):
    return pl.kernel(
        _scatter_body,
        out_type=jax.ShapeDtypeStruct((CAP, 8, 128), jnp.float32),
        mesh=plsc.VectorSubcoreMesh(core_axis_name="c", subcore_axis_name="s"),
        scratch_types=[
            pltpu.VMEM((_SNC, _SCH), jnp.int32),
            pltpu.VMEM((_SNC, _SCH), jnp.int32),
            pltpu.VMEM((_SCH, 8, 128), jnp.float32),
            pltpu.VMEM((_SCH, 8, 128), jnp.float32),
            pltpu.SemaphoreType.DMA,
            pltpu.SemaphoreType.DMA,
            pltpu.SemaphoreType.DMA,
            pltpu.SemaphoreType.DMA,
            pltpu.SemaphoreType.DMA,
        ],
    )(x3, pos)


def _gmm_body(be_ref, xs_ref, wd_ref, bd_ref, wu_ref, bu_ref, ys_ref):
    xs = xs_ref[...].reshape(BM, D)
    h = jnp.dot(xs, wd_ref[0], preferred_element_type=jnp.float32)
    h = jnp.maximum(h + bd_ref[0], 0.0)
    y = jnp.dot(h, wu_ref[0], preferred_element_type=jnp.float32)
    ys_ref[...] = (y + bu_ref[0]) * 0.5


def _gmm(block_expert, xs, Wd, bd, Wu, bu):
    grid_spec = pltpu.PrefetchScalarGridSpec(
        num_scalar_prefetch=1,
        grid=(NB,),
        in_specs=[
            pl.BlockSpec((BM, 8, 128), lambda b, be: (b, 0, 0)),
            pl.BlockSpec((1, D, F), lambda b, be: (be[b], 0, 0)),
            pl.BlockSpec((1, 1, F), lambda b, be: (be[b], 0, 0)),
            pl.BlockSpec((1, F, D), lambda b, be: (be[b], 0, 0)),
            pl.BlockSpec((1, 1, D), lambda b, be: (be[b], 0, 0)),
        ],
        out_specs=pl.BlockSpec((BM, D), lambda b, be: (b, 0)),
    )
    return pl.pallas_call(
        _gmm_body,
        grid_spec=grid_spec,
        out_shape=jax.ShapeDtypeStruct((CAP, D), jnp.float32),
    )(block_expert, xs, Wd, bd.reshape(E, 1, F), Wu, bu.reshape(E, 1, D))


_CCH = 16
_CNC = TW // _CCH


def _combine_body(ys_hbm, pos_hbm, g1_hbm, g2_hbm, out_hbm,
                  p1_v, p2_v, g1_v, g2_v, a0, a1, b0, b1,
                  ga0, ga1, gb0, gb1, w0, w1):
    wid = lax.axis_index("s") * NC + lax.axis_index("c")
    base = wid * TW
    pltpu.sync_copy(pos_hbm.at[pl.ds(base, TW)], p1_v)
    pltpu.sync_copy(pos_hbm.at[pl.ds(N_TOK + base, TW)], p2_v)
    pltpu.sync_copy(g1_hbm.at[pl.ds(base, TW)], g1_v)
    pltpu.sync_copy(g2_hbm.at[pl.ds(base, TW)], g2_v)
    ra = (a0, a1)
    rb = (b0, b1)
    gsa = (ga0, ga1)
    gsb = (gb0, gb1)
    wsem = (w0, w1)
    iota = lax.iota(jnp.int32, L)

    def fire(c, k):
        sl = pl.ds(c * _CCH, _CCH)
        return (pltpu.async_copy(ys_hbm.at[p1_v.at[sl]], ra[k], gsa[k]),
                pltpu.async_copy(ys_hbm.at[p2_v.at[sl]], rb[k], gsb[k]))

    gh = [fire(0, 0), None]
    wh = [None, None]
    for c in range(_CNC):
        k = c & 1
        if c + 1 < _CNC:
            if wh[1 - k] is not None:
                wh[1 - k].wait()
            gh[1 - k] = fire(c + 1, 1 - k)
        gh[k][0].wait()
        gh[k][1].wait()
        g1c = g1_v[pl.ds(c * _CCH, L)]
        g2c = g2_v[pl.ds(c * _CCH, L)]

        def tok(t, carry, k=k, g1c=g1c, g2c=g2c):
            ga = jnp.take(g1c, jnp.zeros((L,), jnp.int32) + t)
            gb = jnp.take(g2c, jnp.zeros((L,), jnp.int32) + t)
            for v in range(D // L):
                sl = pl.ds(v * L, L)
                ra[k][t, sl] = ga * ra[k][t, sl] + gb * rb[k][t, sl]
            return carry

        lax.fori_loop(0, _CCH, tok, 0)
        wh[k] = pltpu.async_copy(
            ra[k], out_hbm.at[pl.ds(base + c * _CCH, _CCH)], wsem[k])
    wh[(_CNC - 1) & 1].wait()
    if wh[_CNC & 1] is not None:
        wh[_CNC & 1].wait()


def _combine(ys, pos, g1, g2):
    return pl.kernel(
        _combine_body,
        out_type=jax.ShapeDtypeStruct((N_TOK, D), jnp.float32),
        mesh=plsc.VectorSubcoreMesh(core_axis_name="c", subcore_axis_name="s"),
        scratch_types=[
            pltpu.VMEM((TW,), jnp.int32),
            pltpu.VMEM((TW,), jnp.int32),
            pltpu.VMEM((TW,), jnp.float32),
            pltpu.VMEM((TW,), jnp.float32),
            pltpu.VMEM((_CCH, D), jnp.float32),
            pltpu.VMEM((_CCH, D), jnp.float32),
            pltpu.VMEM((_CCH, D), jnp.float32),
            pltpu.VMEM((_CCH, D), jnp.float32),
            pltpu.SemaphoreType.DMA,
            pltpu.SemaphoreType.DMA,
            pltpu.SemaphoreType.DMA,
            pltpu.SemaphoreType.DMA,
            pltpu.SemaphoreType.DMA,
            pltpu.SemaphoreType.DMA,
        ],
    )(ys, pos, g1, g2)


def kernel(x, w_gate, Wd, bd, Wu, bu):
    i1, i2, g1, g2, x3 = _gating(x, w_gate)
    hists = _hist(i1, i2)
    counts = jnp.sum(hists, axis=0)
    cap_al = ((counts + BM - 1) // BM) * BM
    base_al = (jnp.cumsum(cap_al) - cap_al).astype(jnp.int32)
    bs = jnp.arange(NB, dtype=jnp.int32) * BM
    block_expert = (jnp.sum(base_al[None, :] <= bs[:, None], axis=1) - 1
                    ).astype(jnp.int32)
    pos = _pos(i1, i2, hists)
    xs = _scatter(x3, pos)
    ys = _gmm(block_expert, xs, Wd, bd, Wu, bu)
    return _combine(ys, pos, g1, g2)
